# Initial kernel scaffold; baseline (speedup 1.0000x reference)
import jax
import jax.numpy as jnp
from jax.experimental import pallas as pl


def kernel(x, params):
    raise NotImplementedError("write your pallas kernel here")


# speedup vs baseline: 4.2902x; 4.2902x over previous
import functools

import jax
import jax.numpy as jnp
import numpy as np
from jax.experimental import pallas as pl

C = 192
K = 9
N = 56 * 56
R = 392


def _mm_kernel(x_ref, w_ref, b_ref, o_ref):
    xr = jnp.maximum(x_ref[:], 0.0)
    dn = (((1,), (1,)), ((), ()))
    o_ref[:] = jax.lax.dot_general(
        xr, w_ref[:], dn, preferred_element_type=jnp.float32) + b_ref[:]


def _mm2c_kernel(x_ref, w_ref, b_ref, o_ref):
    xr = jnp.maximum(x_ref[:], 0.0)
    dn = (((1,), (1,)), ((), ()))
    a = jax.lax.dot_general(xr[:, :256], w_ref[:, :256], dn,
                            preferred_element_type=jnp.float32)
    c = jax.lax.dot_general(xr[:, 256:], w_ref[:, 256:], dn,
                            preferred_element_type=jnp.float32)
    o_ref[:] = (a + c) + b_ref[:]


def _pl_mm(x, w, b):
    body = _mm2c_kernel if x.shape[1] == 2 * C else _mm_kernel
    return pl.pallas_call(
        body, out_shape=jax.ShapeDtypeStruct((x.shape[0], w.shape[0]),
                                             jnp.float32),
    )(x, w, b.reshape(1, -1))


def _knn_kernel(xb_ref, sqc_ref, sqr_ref, xall_ref, mr_ref):
    xb = xb_ref[:]
    dn = (((1,), (1,)), ((), ()))
    e = jax.lax.dot_general(xb, xall_ref[:], dn,
                            preferred_element_type=jnp.float32)
    d = (sqc_ref[:] + sqr_ref[:]) - 2.0 * e
    col = jax.lax.broadcasted_iota(jnp.int32, (R, N), 1)
    acc = jnp.full((R, C), -jnp.inf, jnp.float32)
    gdn = (((1,), (0,)), ((), ()))
    for _ in range(K):
        m = jnp.min(d, axis=1, keepdims=True)
        jm = jnp.min(jnp.where(d == m, col, N), axis=1, keepdims=True)
        oh = (col == jm).astype(jnp.float32)
        xj = jax.lax.dot_general(oh, xall_ref[:], gdn,
                                 preferred_element_type=jnp.float32,
                                 precision=jax.lax.Precision.HIGHEST)
        acc = jnp.maximum(acc, xj)
        d = jnp.where(col == jm, jnp.inf, d)
    mr_ref[:] = acc - xb


def _pl_knn_maxrel(x2, sq):
    return pl.pallas_call(
        _knn_kernel,
        grid=(N // R,),
        in_specs=[pl.BlockSpec((R, C), lambda i: (i, 0)),
                  pl.BlockSpec((R, 1), lambda i: (i, 0)),
                  pl.BlockSpec((1, N), lambda i: (0, 0)),
                  pl.BlockSpec((N, C), lambda i: (0, 0))],
        out_specs=pl.BlockSpec((R, C), lambda i: (i, 0)),
        out_shape=jax.ShapeDtypeStruct((N, C), jnp.float32),
    )(x2, sq.reshape(N, 1), sq.reshape(1, N), x2)


_EYE = {C: np.eye(C, dtype=np.float32),
        2 * C: np.eye(2 * C, dtype=np.float32)}


def _idot(x):
    return jnp.matmul(x, _EYE[x.shape[-1]],
                      precision=jax.lax.Precision.HIGHEST)


def _lorentz_linear(x, p):
    x = _idot(jax.nn.relu(x))
    y = _pl_mm(x[0], p["W"], p["b"])[None]
    y = _idot(y)
    xn = y[..., 1:]
    time = jax.nn.sigmoid(y[..., :1]) * jnp.exp(p["s"]) + 1.1
    denom = jnp.maximum(jnp.sum(xn * xn, axis=-1, keepdims=True), 1e-8)
    sc = (time * time - 1.0) / denom
    return jnp.concatenate([time, xn * jnp.sqrt(sc)], axis=-1)


def _ffn(x, p1, p2):
    t = x
    x = _lorentz_linear(x, p1)
    x = _lorentz_linear(x, p2)
    return x + t


def _graph_conv(x4, k, dilation, p):
    B, c, n, _ = x4.shape
    x = jnp.transpose(x4[..., 0], (0, 2, 1))
    x3 = x4[..., 0]
    sq = jnp.sum(x3 * x3, axis=1)
    max_rel = _pl_knn_maxrel(x[0], sq[0])[None]
    out = _lorentz_linear(jnp.concatenate([x, max_rel], axis=-1), p)
    return jnp.transpose(out, (0, 2, 1))[..., None]


def _conv(x, w, b, stride, pad):
    y = jax.lax.conv_general_dilated(x, w, (stride, stride),
                                     [(pad, pad), (pad, pad)],
                                     dimension_numbers=("NCHW", "OIHW", "NCHW"))
    return y + b[None, :, None, None]


def _bn(x, g, be):
    return x * g[None, :, None, None] + be[None, :, None, None]


def _stem(x, p):
    x = jax.nn.relu(_bn(_conv(x, p["w1"], p["b1"], 2, 1), p["g1"], p["be1"]))
    x = jax.nn.relu(_bn(_conv(x, p["w2"], p["b2"], 2, 1), p["g2"], p["be2"]))
    x = _bn(_conv(x, p["w3"], p["b3"], 1, 1), p["g3"], p["be3"])
    return x


def _grapher(x, k, dilation, p):
    B, c, Hh, Ww = x.shape
    x4 = x.reshape(B, c, Hh * Ww, 1)
    b, cc, h, w = x4.shape
    shortcut = x4
    t = jnp.transpose(x4[..., 0], (0, 2, 1))
    t = _ffn(t, p["fc1_1"], p["fc1_2"])
    t = _idot(t)
    t = t.reshape(b, cc, h, w)
    t = _graph_conv(t, k, dilation, p["gc"])
    t = jnp.transpose(t[..., 0], (0, 2, 1))
    t = _ffn(t, p["fc2_1"], p["fc2_2"])
    t = t.reshape(b, cc, h, w)
    t = t + shortcut
    return t.reshape(B, c, Hh, Ww)


def _vig_block(x, k, dilation, p):
    B, c, Hh, Ww = x.shape
    x = _grapher(x, k, dilation, p)
    t = jnp.transpose(x.reshape(B, c, Hh * Ww), (0, 2, 1))
    t = _ffn(t, p["ffn_1"], p["ffn_2"])
    return t.reshape(B, c, Hh, Ww)


def kernel(x, params):
    h = _stem(x, params["stem"]) + params["pos_embed"]
    for i, bp in enumerate(params["blocks"]):
        h = _vig_block(h, K, 1, bp)
    return h
